# baseline (speedup 1.0000x reference)
import jax
import jax.numpy as jnp
from jax import lax
from jax.experimental import pallas as pl
from jax.experimental.pallas import tpu as pltpu
from jax.experimental.pallas import tpu_sc as plsc

E = 32
D = 64
ED = E * D
MEM = 16384
B = 4096

NC = 2
NS = 16
NW = NC * NS

GCH = 8
B_PER_W = B // NW
NCHUNK = B_PER_W // GCH

_sc_mesh = plsc.VectorSubcoreMesh(core_axis_name="c", subcore_axis_name="s")


def _gather_body(h_hbm, k_hbm, idx_hbm, curh_hbm, curk_hbm,
                 idx0, idx1, hb0, hb1, kb0, kb1, sh0, sh1, sk0, sk1):
    c = lax.axis_index("c")
    s = lax.axis_index("s")
    wid = s * NC + c
    base = wid * B_PER_W
    idxs, hbs, kbs, shs, sks = (idx0, idx1), (hb0, hb1), (kb0, kb1), \
        (sh0, sh1), (sk0, sk1)

    def start(cc, b):
        off = base + cc * GCH
        pltpu.sync_copy(idx_hbm.at[pl.ds(off, GCH)], idxs[b])
        pltpu.async_copy(h_hbm.at[idxs[b]], hbs[b], shs[b])
        pltpu.async_copy(k_hbm.at[idxs[b]], kbs[b], sks[b])

    def wait(b):
        pltpu.make_async_copy(h_hbm.at[idxs[b]], hbs[b], shs[b]).wait()
        pltpu.make_async_copy(k_hbm.at[idxs[b]], kbs[b], sks[b]).wait()

    start(0, 0)

    def outer(g, carry):
        for b in range(2):
            cc = g * 2 + b
            nb = 1 - b

            @pl.when(cc + 1 < NCHUNK)
            def _():
                start(cc + 1, nb)

            wait(b)
            off = base + cc * GCH
            pltpu.sync_copy(hbs[b], curh_hbm.at[pl.ds(off, GCH)])
            pltpu.sync_copy(kbs[b], curk_hbm.at[pl.ds(off, GCH)])
        return carry

    lax.fori_loop(0, NCHUNK // 2, outer, 0)


_gather_call = pl.kernel(
    _gather_body,
    out_type=(
        jax.ShapeDtypeStruct((B, ED), jnp.float32),
        jax.ShapeDtypeStruct((B, ED), jnp.float32),
    ),
    mesh=_sc_mesh,
    compiler_params=pltpu.CompilerParams(needs_layout_passes=False),
    scratch_types=[
        pltpu.VMEM((GCH,), jnp.int32),
        pltpu.VMEM((GCH,), jnp.int32),
        pltpu.VMEM((GCH, ED), jnp.float32),
        pltpu.VMEM((GCH, ED), jnp.float32),
        pltpu.VMEM((GCH, ED), jnp.float32),
        pltpu.VMEM((GCH, ED), jnp.float32),
        pltpu.SemaphoreType.DMA,
        pltpu.SemaphoreType.DMA,
        pltpu.SemaphoreType.DMA,
        pltpu.SemaphoreType.DMA,
    ],
)

BB = 256


def _upd_body(h_ref, k_ref, es_ref, uv_ref, w_ref, out_ref):
    h2 = h_ref[...]
    k2 = k_ref[...]
    es = es_ref[...]
    h3 = h2.reshape(BB, E, D)
    k3 = k2.reshape(BB, E, D)
    gates = jax.nn.sigmoid(jnp.sum(es[:, None, :] * (h3 + k3), axis=2))
    ht = jnp.dot(h2, uv_ref[...], preferred_element_type=jnp.float32)
    sw = jnp.dot(es, w_ref[...], preferred_element_type=jnp.float32)
    ht3 = jnp.maximum(ht.reshape(BB, E, D) + sw[:, None, :], 0.0)
    out_ref[...] = (gates[:, :, None] * ht3).reshape(BB * E, D)


_upd_call = pl.pallas_call(
    _upd_body,
    grid=(B // BB,),
    in_specs=[
        pl.BlockSpec((BB * E, D), lambda i: (i, 0)),
        pl.BlockSpec((BB * E, D), lambda i: (i, 0)),
        pl.BlockSpec((BB, D), lambda i: (i, 0)),
        pl.BlockSpec((D, D), lambda i: (0, 0)),
        pl.BlockSpec((D, D), lambda i: (0, 0)),
    ],
    out_specs=pl.BlockSpec((BB * E, D), lambda i: (i, 0)),
    out_shape=jax.ShapeDtypeStruct((B * E, D), jnp.float32),
)

PR = 32
W_ROWS = MEM // NW
NPH = W_ROWS // PR
NJV = B // 16


def _scatter_body(upd_hbm, idx_hbm, hid_hbm, out_hbm,
                  idx_all, sel0pos, sel0loc, selpos, selloc, rowbuf, hid_buf):
    c = lax.axis_index("c")
    s = lax.axis_index("s")
    wid = s * NC + c
    wbase = wid * W_ROWS
    pltpu.sync_copy(idx_hbm, idx_all)
    lanes = lax.iota(jnp.int32, 16)
    zeros16 = jnp.zeros((16,), jnp.int32)

    def scan0(j, cnt_c):
        v = idx_all[pl.ds(j * 16, 16)]
        m = (v >= wbase) & (v < wbase + W_ROWS)
        pos = cnt_c + plsc.cumsum(m.astype(jnp.int32)) - 1
        plsc.store_scatter(sel0pos, [pos], j * 16 + lanes, mask=m)
        plsc.store_scatter(sel0loc, [pos], v - wbase, mask=m)
        return cnt_c + plsc.all_reduce_population_count(m)

    cnt0 = lax.fori_loop(0, NJV, scan0, zeros16)
    nv0 = (jnp.max(cnt0) + 15) // 16

    def phase(p, carry):
        pbase = wbase + p * PR
        pltpu.sync_copy(hid_hbm.at[pl.ds(pbase, PR)], hid_buf)

        def scan(j, cnt_c):
            u = sel0loc[pl.ds(j * 16, 16)]
            pv = sel0pos[pl.ds(j * 16, 16)]
            m = ((u >= p * PR) & (u < p * PR + PR)
                 & ((j * 16 + lanes) < cnt0))
            pos = cnt_c + plsc.cumsum(m.astype(jnp.int32)) - 1
            plsc.store_scatter(selpos, [pos], pv, mask=m)
            plsc.store_scatter(selloc, [pos], u - p * PR, mask=m)
            return cnt_c + plsc.all_reduce_population_count(m)

        cnt = lax.fori_loop(0, nv0, scan, zeros16)
        nch = (jnp.max(cnt) + 15) // 16

        def chunk(ii, c2):
            gi = selpos[pl.ds(ii * 16, 16)] & (B - 1)
            pltpu.sync_copy(upd_hbm.at[gi], rowbuf)
            tl = selloc[pl.ds(ii * 16, 16)]
            valid = (ii * 16 + lanes) < cnt

            def col(cc, c3):
                for u in range(8):
                    cu = cc * 8 + u
                    csplat = jnp.full((16,), 0, jnp.int32) + cu
                    x = plsc.load_gather(rowbuf, [lanes, csplat])
                    plsc.addupdate_scatter(hid_buf, [tl, csplat], x,
                                           mask=valid)
                return c3

            lax.fori_loop(0, ED // 8, col, 0)
            return c2

        lax.fori_loop(0, nch, chunk, 0)
        pltpu.sync_copy(hid_buf, out_hbm.at[pl.ds(pbase, PR)])
        return carry

    lax.fori_loop(0, NPH, phase, 0)


_scatter_call = pl.kernel(
    _scatter_body,
    out_type=jax.ShapeDtypeStruct((MEM, ED), jnp.float32),
    mesh=_sc_mesh,
    compiler_params=pltpu.CompilerParams(needs_layout_passes=False),
    scratch_types=[
        pltpu.VMEM((B,), jnp.int32),
        pltpu.VMEM((B + 16,), jnp.int32),
        pltpu.VMEM((B + 16,), jnp.int32),
        pltpu.VMEM((B + 16,), jnp.int32),
        pltpu.VMEM((B + 16,), jnp.int32),
        pltpu.VMEM((16, ED), jnp.float32),
        pltpu.VMEM((PR, ED), jnp.float32),
    ],
)

NB = 256


def _norm_body(x_ref, o_ref):
    x = x_ref[...]
    ss = jnp.sum(x * x, axis=1, keepdims=True)
    o_ref[...] = x * lax.rsqrt(jnp.maximum(ss, 1e-12))


_norm_call = pl.pallas_call(
    _norm_body,
    grid=(MEM // NB,),
    in_specs=[pl.BlockSpec((NB * E, D), lambda i: (i, 0))],
    out_specs=pl.BlockSpec((NB * E, D), lambda i: (i, 0)),
    out_shape=jax.ShapeDtypeStruct((MEM * E, D), jnp.float32),
)


def kernel(encoded_sents, indices, hiddens, keys, U, V, W):
    h2 = hiddens.reshape(MEM, ED)
    k2 = keys.reshape(MEM, ED)
    cur_h, cur_k = _gather_call(h2, k2, indices)
    upd = _upd_call(cur_h.reshape(B * E, D), cur_k.reshape(B * E, D),
                    encoded_sents, U + V, W)
    out2 = _scatter_call(upd.reshape(B, ED), indices, h2)
    out = _norm_call(out2.reshape(MEM * E, D))
    return out.reshape(MEM, E, D)


# speedup vs baseline: 1.2769x; 1.2769x over previous
import jax
import jax.numpy as jnp
from jax import lax
from jax.experimental import pallas as pl
from jax.experimental.pallas import tpu as pltpu
from jax.experimental.pallas import tpu_sc as plsc

E = 32
D = 64
MEM = 16384
B = 4096
BT = 4864

NC = 2
NS = 16
NW = NC * NS

NBLK = 128
NSEG = MEM // NBLK
QPW = NSEG // NW
W_ROWS = MEM // NW
NJV = B // 16
GCH = 16
PACK = 16384

_sc_mesh = plsc.VectorSubcoreMesh(core_axis_name="c", subcore_axis_name="s")


def _group_body(idx_hbm, idxp_hbm, posp_hbm, packed_hbm,
                idx_all, valbuf, posbuf, offstage, vstage, pstage):
    c = lax.axis_index("c")
    s = lax.axis_index("s")
    wid = s * NC + c
    wbase = wid * W_ROWS
    pltpu.sync_copy(idx_hbm, idx_all)
    lanes = lax.iota(jnp.int32, 16)
    zeros16 = jnp.zeros((16,), jnp.int32)
    ones16 = zeros16 == zeros16

    def scan_base(j, cnt_c):
        v = idx_all[pl.ds(j * 16, 16)]
        return cnt_c + plsc.all_reduce_population_count(v < wbase)

    off_base = lax.fori_loop(0, NJV, scan_base, zeros16)
    aw = (jnp.max(off_base) + 15) // 16 * 16 + 16 * wid

    def seg(q, cnt_c):
        b0 = wbase + q * NBLK

        def scan(j, cc):
            v = idx_all[pl.ds(j * 16, 16)]
            m = (v >= b0) & (v < b0 + NBLK)
            pos = cc + plsc.cumsum(m.astype(jnp.int32)) - 1
            plsc.store_scatter(valbuf, [pos], v, mask=m)
            plsc.store_scatter(posbuf, [pos], j * 16 + lanes, mask=m)
            return cc + plsc.all_reduce_population_count(m)

        cnt_n = lax.fori_loop(0, NJV, scan, cnt_c)
        packed = (aw + jnp.max(cnt_c)) * PACK + (jnp.max(cnt_n) -
                                                 jnp.max(cnt_c))
        plsc.store_scatter(offstage, [jnp.full((16,), q, jnp.int32)],
                           jnp.full((16,), 0, jnp.int32) + packed,
                           mask=(lanes == 0))
        return cnt_n

    cnt = lax.fori_loop(0, QPW, seg, zeros16)
    pltpu.sync_copy(offstage, packed_hbm.at[pl.ds(wid * 16, 16)])

    plsc.store_scatter(valbuf, [jnp.max(cnt) + lanes], zeros16, mask=ones16)
    plsc.store_scatter(posbuf, [jnp.max(cnt) + lanes], zeros16, mask=ones16)

    nch = (jnp.max(cnt) + GCH - 1) // GCH

    def chunk(cc, carry):
        vstage[...] = valbuf[pl.ds(cc * GCH, GCH)]
        pstage[...] = posbuf[pl.ds(cc * GCH, GCH)]
        dst = aw + cc * GCH
        pltpu.sync_copy(vstage, idxp_hbm.at[pl.ds(dst, GCH)])
        pltpu.sync_copy(pstage, posp_hbm.at[pl.ds(dst, GCH)])
        return carry

    lax.fori_loop(0, nch, chunk, 0)


_group_call = pl.kernel(
    _group_body,
    out_type=(
        jax.ShapeDtypeStruct((BT,), jnp.int32),
        jax.ShapeDtypeStruct((BT,), jnp.int32),
        jax.ShapeDtypeStruct((NW * 16,), jnp.int32),
    ),
    mesh=_sc_mesh,
    compiler_params=pltpu.CompilerParams(needs_layout_passes=False),
    scratch_types=[
        pltpu.VMEM((B,), jnp.int32),
        pltpu.VMEM((B + 16,), jnp.int32),
        pltpu.VMEM((B + 16,), jnp.int32),
        pltpu.VMEM((16,), jnp.int32),
        pltpu.VMEM((GCH,), jnp.int32),
        pltpu.VMEM((GCH,), jnp.int32),
    ],
)

BBK = 128


def _slot_body(idxp_ref, posp_ref, k_hbm, es_ref, w_ref,
               esg_ref, esw_ref, dkg_ref, kbuf, sem):
    i = pl.program_id(0)

    def issue(j, carry):
        r = jnp.clip(idxp_ref[i * BBK + j], 0, MEM - 1)
        pltpu.make_async_copy(k_hbm.at[pl.ds(r, 1)], kbuf.at[pl.ds(j, 1)],
                              sem).start()
        return carry

    lax.fori_loop(0, BBK, issue, 0)

    def pick_es(j, carry):
        p = jnp.clip(posp_ref[i * BBK + j], 0, B - 1)
        esg_ref[pl.ds(j, 1)] = es_ref[pl.ds(p, 1)]
        return carry

    lax.fori_loop(0, BBK, pick_es, 0)
    esg = esg_ref[...]
    esw_ref[...] = jnp.dot(esg, w_ref[...], preferred_element_type=jnp.float32)

    def drain(j, carry):
        pltpu.make_async_copy(k_hbm.at[pl.ds(0, 1)], kbuf.at[pl.ds(j, 1)],
                              sem).wait()
        return carry

    lax.fori_loop(0, BBK, drain, 0)
    dkg_ref[...] = jnp.sum(esg[:, None, :] * kbuf[...], axis=2)


_slot_call = pl.pallas_call(
    _slot_body,
    grid_spec=pltpu.PrefetchScalarGridSpec(
        num_scalar_prefetch=2,
        grid=(BT // BBK,),
        in_specs=[
            pl.BlockSpec(memory_space=pl.ANY),
            pl.BlockSpec((B, D), lambda i, idxp, posp: (0, 0)),
            pl.BlockSpec((D, D), lambda i, idxp, posp: (0, 0)),
        ],
        out_specs=[
            pl.BlockSpec((BBK, D), lambda i, idxp, posp: (i, 0)),
            pl.BlockSpec((BBK, D), lambda i, idxp, posp: (i, 0)),
            pl.BlockSpec((BBK, E), lambda i, idxp, posp: (i, 0)),
        ],
        scratch_shapes=[
            pltpu.VMEM((BBK, E, D), jnp.float32),
            pltpu.SemaphoreType.DMA,
        ],
    ),
    out_shape=(
        jax.ShapeDtypeStruct((BT, D), jnp.float32),
        jax.ShapeDtypeStruct((BT, D), jnp.float32),
        jax.ShapeDtypeStruct((BT, E), jnp.float32),
    ),
)

CH = 32


def _apply_body(packed_ref, idxp_ref, hid_ref, esg_hbm, esw_hbm, dkg_hbm,
                uv_ref, o_ref, esg_c, esw_c, dkg_c, hstage, ustage,
                s0, s1, s2):
    i = pl.program_id(0)
    o_ref[...] = hid_ref[...]
    packed = packed_ref[(i // QPW) * 16 + i % QPW]
    start = packed // PACK
    n = packed % PACK
    nch = (n + CH - 1) // CH

    def chunk_body(cc, carry):
        base = start + cc * CH
        c0 = pltpu.make_async_copy(esg_hbm.at[pl.ds(base, CH)], esg_c, s0)
        c1 = pltpu.make_async_copy(esw_hbm.at[pl.ds(base, CH)], esw_c, s1)
        c2 = pltpu.make_async_copy(dkg_hbm.at[pl.ds(base, CH)], dkg_c, s2)
        c0.start(); c1.start(); c2.start()

        def pick_h(j, carry2):
            t = idxp_ref[base + j] - i * NBLK
            t = jnp.clip(t, 0, NBLK - 1)
            hstage[pl.ds(j, 1)] = hid_ref[pl.ds(t, 1)]
            return carry2

        lax.fori_loop(0, CH, pick_h, 0)
        c0.wait(); c1.wait(); c2.wait()

        h3 = hstage[...]
        esg = esg_c[...]
        esw = esw_c[...]
        dkg = dkg_c[...]
        gates = jax.nn.sigmoid(jnp.sum(h3 * esg[:, None, :], axis=2) + dkg)
        mm = jnp.dot(h3.reshape(CH * E, D), uv_ref[...],
                     preferred_element_type=jnp.float32)
        cand = jnp.maximum(mm.reshape(CH, E, D) + esw[:, None, :], 0.0)
        ustage[...] = gates[:, :, None] * cand

        cnt = jnp.minimum(n - cc * CH, CH)

        def row(j, carry2):
            t = idxp_ref[base + j] - i * NBLK
            o_ref[pl.ds(t, 1)] = o_ref[pl.ds(t, 1)] + ustage[pl.ds(j, 1)]
            return carry2

        return lax.fori_loop(0, cnt, row, carry)

    lax.fori_loop(0, nch, chunk_body, 0)

    x = o_ref[...]
    ss = jnp.maximum(jnp.sum(x * x, axis=2, keepdims=True), 1e-12)
    o_ref[...] = x * lax.rsqrt(ss)


_apply_call = pl.pallas_call(
    _apply_body,
    grid_spec=pltpu.PrefetchScalarGridSpec(
        num_scalar_prefetch=2,
        grid=(NSEG,),
        in_specs=[
            pl.BlockSpec((NBLK, E, D), lambda i, packed, idxp: (i, 0, 0)),
            pl.BlockSpec(memory_space=pl.ANY),
            pl.BlockSpec(memory_space=pl.ANY),
            pl.BlockSpec(memory_space=pl.ANY),
            pl.BlockSpec((D, D), lambda i, packed, idxp: (0, 0)),
        ],
        out_specs=pl.BlockSpec((NBLK, E, D), lambda i, packed, idxp: (i, 0, 0)),
        scratch_shapes=[
            pltpu.VMEM((CH, D), jnp.float32),
            pltpu.VMEM((CH, D), jnp.float32),
            pltpu.VMEM((CH, E), jnp.float32),
            pltpu.VMEM((CH, E, D), jnp.float32),
            pltpu.VMEM((CH, E, D), jnp.float32),
            pltpu.SemaphoreType.DMA,
            pltpu.SemaphoreType.DMA,
            pltpu.SemaphoreType.DMA,
        ],
    ),
    out_shape=jax.ShapeDtypeStruct((MEM, E, D), jnp.float32),
)


def kernel(encoded_sents, indices, hiddens, keys, U, V, W):
    idxp, posp, packed = _group_call(indices)
    es_g, esw_g, dk_g = _slot_call(idxp, posp, keys, encoded_sents, W)
    return _apply_call(packed, idxp, hiddens, es_g, esw_g, dk_g, U + V)


# speedup vs baseline: 1.7235x; 1.3498x over previous
import jax
import jax.numpy as jnp
from jax import lax
from jax.experimental import pallas as pl
from jax.experimental.pallas import tpu as pltpu
from jax.experimental.pallas import tpu_sc as plsc

E = 32
D = 64
MEM = 16384
B = 4096
BT = 4864

NC = 2
NS = 16
NW = NC * NS

NBLK = 128
NSEG = MEM // NBLK
QPW = NSEG // NW
W_ROWS = MEM // NW
NJV = B // 16
GCH = 16
PACK = 16384

_sc_mesh = plsc.VectorSubcoreMesh(core_axis_name="c", subcore_axis_name="s")


def _group_body(idx_hbm, idxp_hbm, posp_hbm, packed_hbm,
                idx_all, valbuf, posbuf, offstage, vstage, pstage):
    c = lax.axis_index("c")
    s = lax.axis_index("s")
    wid = s * NC + c
    wbase = wid * W_ROWS
    pltpu.sync_copy(idx_hbm, idx_all)
    lanes = lax.iota(jnp.int32, 16)
    zeros16 = jnp.zeros((16,), jnp.int32)
    ones16 = zeros16 == zeros16

    def scan_base(j, cnt_c):
        v = idx_all[pl.ds(j * 16, 16)]
        return cnt_c + plsc.all_reduce_population_count(v < wbase)

    off_base = lax.fori_loop(0, NJV, scan_base, zeros16)
    aw = (jnp.max(off_base) + 15) // 16 * 16 + 16 * wid

    def seg(q, cnt_c):
        b0 = wbase + q * NBLK

        def scan(j, cc):
            v = idx_all[pl.ds(j * 16, 16)]
            m = (v >= b0) & (v < b0 + NBLK)
            pos = cc + plsc.cumsum(m.astype(jnp.int32)) - 1
            plsc.store_scatter(valbuf, [pos], v, mask=m)
            plsc.store_scatter(posbuf, [pos], j * 16 + lanes, mask=m)
            return cc + plsc.all_reduce_population_count(m)

        cnt_n = lax.fori_loop(0, NJV, scan, cnt_c)
        packed = (aw + jnp.max(cnt_c)) * PACK + (jnp.max(cnt_n) -
                                                 jnp.max(cnt_c))
        plsc.store_scatter(offstage, [jnp.full((16,), q, jnp.int32)],
                           jnp.full((16,), 0, jnp.int32) + packed,
                           mask=(lanes == 0))
        return cnt_n

    cnt = lax.fori_loop(0, QPW, seg, zeros16)
    pltpu.sync_copy(offstage, packed_hbm.at[pl.ds(wid * 16, 16)])

    plsc.store_scatter(valbuf, [jnp.max(cnt) + lanes], zeros16, mask=ones16)
    plsc.store_scatter(posbuf, [jnp.max(cnt) + lanes], zeros16, mask=ones16)

    nch = (jnp.max(cnt) + GCH - 1) // GCH

    def chunk(cc, carry):
        vstage[...] = valbuf[pl.ds(cc * GCH, GCH)]
        pstage[...] = posbuf[pl.ds(cc * GCH, GCH)]
        dst = aw + cc * GCH
        pltpu.sync_copy(vstage, idxp_hbm.at[pl.ds(dst, GCH)])
        pltpu.sync_copy(pstage, posp_hbm.at[pl.ds(dst, GCH)])
        return carry

    lax.fori_loop(0, nch, chunk, 0)


_group_call = pl.kernel(
    _group_body,
    out_type=(
        jax.ShapeDtypeStruct((BT,), jnp.int32),
        jax.ShapeDtypeStruct((BT,), jnp.int32),
        jax.ShapeDtypeStruct((NW * 16,), jnp.int32),
    ),
    mesh=_sc_mesh,
    compiler_params=pltpu.CompilerParams(needs_layout_passes=False),
    scratch_types=[
        pltpu.VMEM((B,), jnp.int32),
        pltpu.VMEM((B + 16,), jnp.int32),
        pltpu.VMEM((B + 16,), jnp.int32),
        pltpu.VMEM((16,), jnp.int32),
        pltpu.VMEM((GCH,), jnp.int32),
        pltpu.VMEM((GCH,), jnp.int32),
    ],
)


def _esw_body(es_ref, w_ref, o_ref):
    o_ref[...] = jnp.dot(es_ref[...], w_ref[...],
                         preferred_element_type=jnp.float32)


_esw_call = pl.pallas_call(
    _esw_body,
    grid=(1,),
    in_specs=[
        pl.BlockSpec((B, D), lambda i: (0, 0)),
        pl.BlockSpec((D, D), lambda i: (0, 0)),
    ],
    out_specs=pl.BlockSpec((B, D), lambda i: (0, 0)),
    out_shape=jax.ShapeDtypeStruct((B, D), jnp.float32),
)

CH = 32


def _apply_body(packed_ref, idxp_ref, posp_ref, hid_ref, key_ref, es_ref,
                esw_ref, uv_ref, o_ref, hstage, kstage, estage, ewstage,
                ustage):
    i = pl.program_id(0)
    o_ref[...] = hid_ref[...]
    packed = packed_ref[(i // QPW) * 16 + i % QPW]
    start = packed // PACK
    n = packed % PACK
    nch = (n + CH - 1) // CH

    def chunk_body(cc, carry):
        base = start + cc * CH

        def pick(j, carry2):
            t = jnp.clip(idxp_ref[base + j] - i * NBLK, 0, NBLK - 1)
            p = jnp.clip(posp_ref[base + j], 0, B - 1)
            hstage[pl.ds(j, 1)] = hid_ref[pl.ds(t, 1)]
            kstage[pl.ds(j, 1)] = key_ref[pl.ds(t, 1)]
            estage[pl.ds(j, 1)] = es_ref[pl.ds(p, 1)]
            ewstage[pl.ds(j, 1)] = esw_ref[pl.ds(p, 1)]
            return carry2

        lax.fori_loop(0, CH, pick, 0)

        h3 = hstage[...]
        k3 = kstage[...]
        esg = estage[...]
        esw = ewstage[...]
        gates = jax.nn.sigmoid(
            jnp.sum((h3 + k3) * esg[:, None, :], axis=2))
        mm = jnp.dot(h3.reshape(CH * E, D), uv_ref[...],
                     preferred_element_type=jnp.float32)
        cand = jnp.maximum(mm.reshape(CH, E, D) + esw[:, None, :], 0.0)
        ustage[...] = gates[:, :, None] * cand

        cnt = jnp.minimum(n - cc * CH, CH)

        def row(j, carry2):
            t = idxp_ref[base + j] - i * NBLK
            o_ref[pl.ds(t, 1)] = o_ref[pl.ds(t, 1)] + ustage[pl.ds(j, 1)]
            return carry2

        return lax.fori_loop(0, cnt, row, carry)

    lax.fori_loop(0, nch, chunk_body, 0)

    x = o_ref[...]
    ss = jnp.maximum(jnp.sum(x * x, axis=2, keepdims=True), 1e-12)
    o_ref[...] = x * lax.rsqrt(ss)


_apply_call = pl.pallas_call(
    _apply_body,
    grid_spec=pltpu.PrefetchScalarGridSpec(
        num_scalar_prefetch=3,
        grid=(NSEG,),
        in_specs=[
            pl.BlockSpec((NBLK, E, D), lambda i, pk, ix, ps: (i, 0, 0)),
            pl.BlockSpec((NBLK, E, D), lambda i, pk, ix, ps: (i, 0, 0)),
            pl.BlockSpec((B, D), lambda i, pk, ix, ps: (0, 0)),
            pl.BlockSpec((B, D), lambda i, pk, ix, ps: (0, 0)),
            pl.BlockSpec((D, D), lambda i, pk, ix, ps: (0, 0)),
        ],
        out_specs=pl.BlockSpec((NBLK, E, D),
                               lambda i, pk, ix, ps: (i, 0, 0)),
        scratch_shapes=[
            pltpu.VMEM((CH, E, D), jnp.float32),
            pltpu.VMEM((CH, E, D), jnp.float32),
            pltpu.VMEM((CH, D), jnp.float32),
            pltpu.VMEM((CH, D), jnp.float32),
            pltpu.VMEM((CH, E, D), jnp.float32),
        ],
    ),
    out_shape=jax.ShapeDtypeStruct((MEM, E, D), jnp.float32),
)


def kernel(encoded_sents, indices, hiddens, keys, U, V, W):
    idxp, posp, packed = _group_call(indices)
    esw = _esw_call(encoded_sents, W)
    return _apply_call(packed, idxp, posp, hiddens, keys,
                       encoded_sents, esw, U + V)


# speedup vs baseline: 1.8388x; 1.0669x over previous
import jax
import jax.numpy as jnp
from jax import lax
from jax.experimental import pallas as pl
from jax.experimental.pallas import tpu as pltpu
from jax.experimental.pallas import tpu_sc as plsc

E = 32
D = 64
MEM = 16384
B = 4096
BT = 4864

NC = 2
NS = 16
NW = NC * NS

NBLK = 256
NSEG = MEM // NBLK
QPW = NSEG // NW
W_ROWS = MEM // NW
NJV = B // 16
GCH = 16
PACK = 16384

_sc_mesh = plsc.VectorSubcoreMesh(core_axis_name="c", subcore_axis_name="s")


def _group_body(idx_hbm, idxp_hbm, posp_hbm, packed_hbm,
                idx_all, valbuf, posbuf, offstage, vstage, pstage):
    c = lax.axis_index("c")
    s = lax.axis_index("s")
    wid = s * NC + c
    wbase = wid * W_ROWS
    pltpu.sync_copy(idx_hbm, idx_all)
    lanes = lax.iota(jnp.int32, 16)
    zeros16 = jnp.zeros((16,), jnp.int32)
    ones16 = zeros16 == zeros16

    def scan_base(j, cnt_c):
        v = idx_all[pl.ds(j * 16, 16)]
        return cnt_c + plsc.all_reduce_population_count(v < wbase)

    off_base = lax.fori_loop(0, NJV, scan_base, zeros16)
    aw = (jnp.max(off_base) + 15) // 16 * 16 + 16 * wid

    def seg(q, cnt_c):
        b0 = wbase + q * NBLK

        def scan(j, cc):
            v = idx_all[pl.ds(j * 16, 16)]
            m = (v >= b0) & (v < b0 + NBLK)
            pos = cc + plsc.cumsum(m.astype(jnp.int32)) - 1
            plsc.store_scatter(valbuf, [pos], v, mask=m)
            plsc.store_scatter(posbuf, [pos], j * 16 + lanes, mask=m)
            return cc + plsc.all_reduce_population_count(m)

        cnt_n = lax.fori_loop(0, NJV, scan, cnt_c)
        packed = (aw + jnp.max(cnt_c)) * PACK + (jnp.max(cnt_n) -
                                                 jnp.max(cnt_c))
        plsc.store_scatter(offstage, [jnp.full((16,), q, jnp.int32)],
                           jnp.full((16,), 0, jnp.int32) + packed,
                           mask=(lanes == 0))
        return cnt_n

    cnt = lax.fori_loop(0, QPW, seg, zeros16)
    pltpu.sync_copy(offstage, packed_hbm.at[pl.ds(wid * 16, 16)])

    plsc.store_scatter(valbuf, [jnp.max(cnt) + lanes], zeros16, mask=ones16)
    plsc.store_scatter(posbuf, [jnp.max(cnt) + lanes], zeros16, mask=ones16)

    nch = (jnp.max(cnt) + GCH - 1) // GCH

    def chunk(cc, carry):
        vstage[...] = valbuf[pl.ds(cc * GCH, GCH)]
        pstage[...] = posbuf[pl.ds(cc * GCH, GCH)]
        dst = aw + cc * GCH
        pltpu.sync_copy(vstage, idxp_hbm.at[pl.ds(dst, GCH)])
        pltpu.sync_copy(pstage, posp_hbm.at[pl.ds(dst, GCH)])
        return carry

    lax.fori_loop(0, nch, chunk, 0)


_group_call = pl.kernel(
    _group_body,
    out_type=(
        jax.ShapeDtypeStruct((BT,), jnp.int32),
        jax.ShapeDtypeStruct((BT,), jnp.int32),
        jax.ShapeDtypeStruct((NW * 16,), jnp.int32),
    ),
    mesh=_sc_mesh,
    compiler_params=pltpu.CompilerParams(needs_layout_passes=False),
    scratch_types=[
        pltpu.VMEM((B,), jnp.int32),
        pltpu.VMEM((B + 16,), jnp.int32),
        pltpu.VMEM((B + 16,), jnp.int32),
        pltpu.VMEM((16,), jnp.int32),
        pltpu.VMEM((GCH,), jnp.int32),
        pltpu.VMEM((GCH,), jnp.int32),
    ],
)

CH = 32


def _apply_body(packed_ref, idxp_ref, posp_ref, hid_ref, key_ref, es_ref,
                w_ref, uv_ref, o_ref, hstage, kstage, estage, ustage):
    i = pl.program_id(0)
    o_ref[...] = hid_ref[...]
    packed = packed_ref[(i // QPW) * 16 + i % QPW]
    start = packed // PACK
    n = packed % PACK
    nch = (n + CH - 1) // CH

    def chunk_body(cc, carry):
        base = start + cc * CH

        def pick(j, carry2):
            t = jnp.clip(idxp_ref[base + j] - i * NBLK, 0, NBLK - 1)
            p = jnp.clip(posp_ref[base + j], 0, B - 1)
            hstage[pl.ds(j, 1)] = hid_ref[pl.ds(t, 1)]
            kstage[pl.ds(j, 1)] = key_ref[pl.ds(t, 1)]
            estage[pl.ds(j, 1)] = es_ref[pl.ds(p, 1)]
            return carry2

        lax.fori_loop(0, CH, pick, 0)

        h3 = hstage[...]
        k3 = kstage[...]
        esg = estage[...]
        esw = jnp.dot(esg, w_ref[...], preferred_element_type=jnp.float32)
        gates = jax.nn.sigmoid(
            jnp.sum((h3 + k3) * esg[:, None, :], axis=2))
        mm = jnp.dot(h3.reshape(CH * E, D), uv_ref[...],
                     preferred_element_type=jnp.float32)
        cand = jnp.maximum(mm.reshape(CH, E, D) + esw[:, None, :], 0.0)
        ustage[...] = gates[:, :, None] * cand

        cnt = jnp.minimum(n - cc * CH, CH)

        def row(j, carry2):
            t = idxp_ref[base + j] - i * NBLK
            o_ref[pl.ds(t, 1)] = o_ref[pl.ds(t, 1)] + ustage[pl.ds(j, 1)]
            return carry2

        return lax.fori_loop(0, cnt, row, carry)

    lax.fori_loop(0, nch, chunk_body, 0)

    x = o_ref[...]
    ss = jnp.maximum(jnp.sum(x * x, axis=2, keepdims=True), 1e-12)
    o_ref[...] = x * lax.rsqrt(ss)


_apply_call = pl.pallas_call(
    _apply_body,
    grid_spec=pltpu.PrefetchScalarGridSpec(
        num_scalar_prefetch=3,
        grid=(NSEG,),
        in_specs=[
            pl.BlockSpec((NBLK, E, D), lambda i, pk, ix, ps: (i, 0, 0)),
            pl.BlockSpec((NBLK, E, D), lambda i, pk, ix, ps: (i, 0, 0)),
            pl.BlockSpec((B, D), lambda i, pk, ix, ps: (0, 0)),
            pl.BlockSpec((D, D), lambda i, pk, ix, ps: (0, 0)),
            pl.BlockSpec((D, D), lambda i, pk, ix, ps: (0, 0)),
        ],
        out_specs=pl.BlockSpec((NBLK, E, D),
                               lambda i, pk, ix, ps: (i, 0, 0)),
        scratch_shapes=[
            pltpu.VMEM((CH, E, D), jnp.float32),
            pltpu.VMEM((CH, E, D), jnp.float32),
            pltpu.VMEM((CH, D), jnp.float32),
            pltpu.VMEM((CH, E, D), jnp.float32),
        ],
    ),
    out_shape=jax.ShapeDtypeStruct((MEM, E, D), jnp.float32),
)


def kernel(encoded_sents, indices, hiddens, keys, U, V, W):
    idxp, posp, packed = _group_call(indices)
    return _apply_call(packed, idxp, posp, hiddens, keys,
                       encoded_sents, W, U + V)
